# Initial kernel scaffold; baseline (speedup 1.0000x reference)
import jax
import jax.numpy as jnp
from jax.experimental import pallas as pl


def kernel(x, edge_index, W1, b1, W2, b2):
    raise NotImplementedError("write your pallas kernel here")


# speedup vs baseline: 5.5877x; 5.5877x over previous
import functools

import jax
import jax.numpy as jnp
from jax import lax
from jax.experimental import pallas as pl
from jax.experimental.pallas import tpu as pltpu
from jax.experimental.pallas import tpu_sc as plsc

N = 10000
D = 256
DH = 128
CB = 64
NCB = D // CB
E = 160000
K_HOPS = 3
ALPHA = 0.5

NC = 2
NS = 16
L = 16

NP = 10240
RPT = NP // NS
RCH = 128
NRCH = RPT // RCH

EPT = 10240
EP = EPT * NS
ECH = 128
NECH = EPT // ECH


def _rsqrt16(v):
    i = lax.bitcast_convert_type(v, jnp.int32)
    i = jnp.int32(0x5F3759DF) - (i >> 1)
    y = lax.bitcast_convert_type(i, jnp.float32)
    for _ in range(3):
        y = y * (1.5 - 0.5 * v * y * y)
    return y


def _sc_body(x_hbm, src_hbm, dst_hbm, state_hbm,
             s_sh, deg_sh,
             src_v, dst_v, gb0, gb1, ones_v, xch, dch, zbuf,
             sem0, sem1):
    c = lax.axis_index("c")
    s = lax.axis_index("s")
    row0 = s * RPT
    erow0 = s * NECH

    pltpu.sync_copy(src_hbm.at[pl.ds(erow0, NECH)], src_v)
    pltpu.sync_copy(dst_hbm.at[pl.ds(erow0, NECH)], dst_v)

    goff0 = (c * (NCB // NC)) * NP

    @pl.loop(0, NECH)
    def _(j):
        for t in range(ECH // L):
            sl = pl.ds(t * L, L)
            src_v[j, sl] = src_v[j, sl] + goff0

    @pl.loop(0, ECH)
    def _(i):
        ones_v[i] = jnp.full((L,), 1.0, jnp.float32)
        for t in range(CB // L):
            zbuf[i, pl.ds(t * L, L)] = jnp.zeros((L,), jnp.float32)

    for k in range(NRCH):
        r0 = row0 + k * RCH
        pltpu.sync_copy(ones_v, deg_sh.at[pl.ds(r0, RCH)])
    plsc.subcore_barrier()

    @pl.loop(0, NECH)
    def _(j):
        pltpu.sync_copy(ones_v, deg_sh.at[dst_v.at[j]], add=True)
    plsc.subcore_barrier()

    for k in range(NRCH):
        r0 = row0 + k * RCH
        pltpu.sync_copy(deg_sh.at[pl.ds(r0, RCH)], dch)

        @pl.loop(0, RCH)
        def _(i):
            dch[i] = _rsqrt16(dch[i])

        pltpu.sync_copy(dch, deg_sh.at[pl.ds(r0, RCH)])
    plsc.subcore_barrier()

    for bi in range(NCB // NC):
        goff = goff0 + bi * NP

        if bi > 0:
            @pl.loop(0, NECH)
            def _(j):
                for t in range(ECH // L):
                    sl = pl.ds(t * L, L)
                    src_v[j, sl] = src_v[j, sl] + NP

        for k in range(NRCH):
            r0 = row0 + k * RCH
            gr0 = goff + r0
            pltpu.sync_copy(deg_sh.at[pl.ds(r0, RCH)], dch)
            pltpu.sync_copy(x_hbm.at[pl.ds(gr0, RCH)], xch)

            @pl.loop(0, RCH)
            def _(i):
                y = dch[i]
                for t in range(CB // L):
                    sl = pl.ds(t * L, L)
                    gb0[i, sl] = y * xch[i, sl]

            pltpu.sync_copy(gb0, state_hbm.at[pl.ds(gr0, RCH)])
            pltpu.sync_copy(zbuf, s_sh.at[pl.ds(r0, RCH)])
        plsc.subcore_barrier()

        for hop in range(K_HOPS):
            last = hop == K_HOPS - 1

            @pl.loop(0, NECH, step=2)
            def _(j):
                c0 = pltpu.async_copy(state_hbm.at[src_v.at[j]], gb0, sem0)
                c1 = pltpu.async_copy(state_hbm.at[src_v.at[j + 1]], gb1, sem1)
                c0.wait()
                pltpu.sync_copy(gb0, s_sh.at[dst_v.at[j]], add=True)
                c1.wait()
                pltpu.sync_copy(gb1, s_sh.at[dst_v.at[j + 1]], add=True)

            plsc.subcore_barrier()

            for k in range(NRCH):
                r0 = row0 + k * RCH
                gr0 = goff + r0
                pltpu.sync_copy(s_sh.at[pl.ds(r0, RCH)], gb0)
                pltpu.sync_copy(state_hbm.at[pl.ds(gr0, RCH)], gb1)
                pltpu.sync_copy(x_hbm.at[pl.ds(gr0, RCH)], xch)
                pltpu.sync_copy(deg_sh.at[pl.ds(r0, RCH)], dch)

                @pl.loop(0, RCH)
                def _(i):
                    b = dch[i]
                    for t in range(CB // L):
                        sl = pl.ds(t * L, L)
                        h = (1.0 - ALPHA) * b * (gb0[i, sl] + gb1[i, sl]) \
                            + ALPHA * xch[i, sl]
                        gb0[i, sl] = h if last else b * h

                pltpu.sync_copy(gb0, state_hbm.at[pl.ds(gr0, RCH)])
                if not last:
                    pltpu.sync_copy(zbuf, s_sh.at[pl.ds(r0, RCH)])
            plsc.subcore_barrier()


@jax.jit
def _sc_propagate(x_flat, src_flat, dst_flat):
    mesh = plsc.VectorSubcoreMesh(core_axis_name="c", subcore_axis_name="s",
                                  num_cores=NC, num_subcores=NS)
    return pl.kernel(
        _sc_body,
        out_type=jax.ShapeDtypeStruct((NCB * NP, CB), jnp.float32),
        mesh=mesh,
        compiler_params=pltpu.CompilerParams(use_tc_tiling_on_sc=False),
        scratch_types=[
            pltpu.VMEM_SHARED((NP, CB), jnp.float32),
            pltpu.VMEM_SHARED((NP, L), jnp.float32),
            pltpu.VMEM((NECH, ECH), jnp.int32),
            pltpu.VMEM((NECH, ECH), jnp.int32),
            pltpu.VMEM((ECH, CB), jnp.float32),
            pltpu.VMEM((ECH, CB), jnp.float32),
            pltpu.VMEM((ECH, L), jnp.float32),
            pltpu.VMEM((RCH, CB), jnp.float32),
            pltpu.VMEM((RCH, L), jnp.float32),
            pltpu.VMEM((RCH, CB), jnp.float32),
            pltpu.SemaphoreType.DMA,
            pltpu.SemaphoreType.DMA,
        ],
    )(x_flat, src_flat, dst_flat)


def _mlp_body(h_ref, w1_ref, b1_ref, w2_ref, b2_ref, emb_ref, log_ref):
    hb = jnp.maximum(h_ref[...], 0.0)
    e = jnp.dot(hb, w1_ref[...], preferred_element_type=jnp.float32) + b1_ref[...]
    emb_ref[...] = e
    log_ref[...] = jnp.dot(jnp.maximum(e, 0.0), w2_ref[...],
                           preferred_element_type=jnp.float32) + b2_ref[...]


@jax.jit
def _mlp(h, w1t, b1r, w2t, b2r):
    blk = 1024
    grid = (NP // blk,)
    return pl.pallas_call(
        _mlp_body,
        grid=grid,
        in_specs=[
            pl.BlockSpec((blk, D), lambda i: (i, 0)),
            pl.BlockSpec((D, D), lambda i: (0, 0)),
            pl.BlockSpec((1, D), lambda i: (0, 0)),
            pl.BlockSpec((D, DH), lambda i: (0, 0)),
            pl.BlockSpec((1, DH), lambda i: (0, 0)),
        ],
        out_specs=[
            pl.BlockSpec((blk, D), lambda i: (i, 0)),
            pl.BlockSpec((blk, DH), lambda i: (i, 0)),
        ],
        out_shape=[
            jax.ShapeDtypeStruct((NP, D), jnp.float32),
            jax.ShapeDtypeStruct((NP, DH), jnp.float32),
        ],
    )(h, w1t, b1r, w2t, b2r)


def kernel(x, edge_index, W1, b1, W2, b2):
    src = edge_index[0].astype(jnp.int32)
    dst = edge_index[1].astype(jnp.int32)
    pad = jnp.full((EP - E,), NP - 1, jnp.int32)
    src_flat = jnp.concatenate([src, pad]).reshape(NS * NECH, ECH)
    dst_flat = jnp.concatenate([dst, pad]).reshape(NS * NECH, ECH)

    xp = jnp.pad(x, ((0, NP - N), (0, 0)))
    x_flat = xp.reshape(NP, NCB, CB).transpose(1, 0, 2).reshape(NCB * NP, CB)

    state = _sc_propagate(x_flat, src_flat, dst_flat)
    h = state.reshape(NCB, NP, CB).transpose(1, 0, 2).reshape(NP, D)

    w2t = jnp.pad(W2, ((0, DH - W2.shape[0]), (0, 0))).T
    b2r = jnp.pad(b2, (0, DH - b2.shape[0])).reshape(1, DH)
    emb, logp = _mlp(h, W1.T, b1.reshape(1, D), w2t, b2r)
    return emb[:N], logp[:N, :40]
